# baseline (speedup 1.0000x reference)
import functools

import jax
import jax.numpy as jnp
from jax import lax
from jax.experimental import pallas as pl
from jax.experimental.pallas import tpu as pltpu
from jax.experimental.pallas import tpu_sc as plsc

N = 10000
E = 320000
D = 128
NC = 2
NS = 16
NW = NC * NS
CH = 128
NCH = 80
EP = NW * NCH * CH
NP = 10112
RPT = NP // NS
HPT = 632
NR = 2
NI = 4

_mesh = plsc.VectorSubcoreMesh(core_axis_name="c", subcore_axis_name="s")


@functools.partial(
    pl.kernel,
    mesh=_mesh,
    out_type=(jax.ShapeDtypeStruct((NC, NP, D), jnp.float32),
              jax.ShapeDtypeStruct((NW, NP), jnp.float32),
              jax.ShapeDtypeStruct((NW, NCH, CH, D), jnp.float32)),
    scratch_types=[
        pltpu.VMEM((NI, CH), jnp.int32),
        pltpu.VMEM((NR, CH, D), jnp.float32),
        pltpu.VMEM((NP,), jnp.float32),
        pltpu.VMEM_SHARED((NP, D), jnp.float32),
        pltpu.SemaphoreType.DMA((NI,)),
        pltpu.SemaphoreType.DMA((NR,)),
        pltpu.SemaphoreType.DMA((NR,)),
    ],
    compiler_params=pltpu.CompilerParams(needs_layout_passes=False),
)
def _sc_aggregate(h_hbm, src_hbm, dst_hbm, part_hbm, cnt_hbm, msg_hbm,
                  idx_v, rows_v, cnt_v, big_sh, isem, gsem, ssem):
    cid = lax.axis_index("c")
    sid = lax.axis_index("s")
    wid = cid * NS + sid

    zeros = jnp.zeros((16,), jnp.float32)
    ones = jnp.ones((16,), jnp.float32)

    def _zero_slot0():
        def _zero_row(r, carry):
            for k in range(D // 16):
                rows_v[0, r, pl.ds(k * 16, 16)] = zeros
            return carry
        lax.fori_loop(0, CH, _zero_row, 0)

    def _zero_cnt(i, carry):
        cnt_v[pl.ds(i * 16, 16)] = zeros
        return carry

    lax.fori_loop(0, NP // 16, _zero_cnt, 0)

    @pl.when(sid < NS - 1)
    def _():
        pltpu.sync_copy(h_hbm.at[pl.ds(sid * HPT, HPT)],
                        big_sh.at[pl.ds(sid * HPT, HPT)])

    @pl.when(sid == NS - 1)
    def _():
        pltpu.sync_copy(h_hbm.at[pl.ds((NS - 1) * HPT, N - (NS - 1) * HPT)],
                        big_sh.at[pl.ds((NS - 1) * HPT, N - (NS - 1) * HPT)])

    plsc.subcore_barrier()

    def _load_start(tab, j, i):
        pltpu.make_async_copy(tab.at[wid].at[j], idx_v.at[i], isem.at[i]).start()

    def _load_wait(tab, j, i):
        pltpu.make_async_copy(tab.at[wid].at[j], idx_v.at[i], isem.at[i]).wait()

    def _ga_start(j, i, r):
        pltpu.make_async_copy(
            big_sh.at[idx_v.at[i]], rows_v.at[r], gsem.at[r]).start()

    def _ga_wait(j, i, r):
        pltpu.make_async_copy(
            big_sh.at[idx_v.at[i]], rows_v.at[r], gsem.at[r]).wait()

    def _wa_start(j, i, r):
        pltpu.make_async_copy(
            rows_v.at[r], msg_hbm.at[wid].at[j], ssem.at[r]).start()

    def _wa_wait(j, i, r):
        pltpu.make_async_copy(
            rows_v.at[r], msg_hbm.at[wid].at[j], ssem.at[r]).wait()

    def _hist_a(i):
        for k in range(CH // 16):
            idx = idx_v[i, pl.ds(k * 16, 16)]
            plsc.addupdate_scatter(cnt_v, [idx], ones)

    def _nohist(i):
        del i

    def _gb_start(j, i, r):
        pltpu.make_async_copy(
            msg_hbm.at[wid].at[j], rows_v.at[r], gsem.at[r]).start()

    def _gb_wait(j, i, r):
        pltpu.make_async_copy(
            msg_hbm.at[wid].at[j], rows_v.at[r], gsem.at[r]).wait()

    def _sb_start(j, i, r):
        pltpu.make_async_copy(
            rows_v.at[r], big_sh.at[idx_v.at[i]], ssem.at[r]).start(add=True)

    def _sb_wait(j, i, r):
        pltpu.make_async_copy(
            rows_v.at[r], big_sh.at[idx_v.at[i]], ssem.at[r]).wait()

    def _phase(tab, hist, prod_start, prod_wait, cons_start, cons_wait):
        def _body(j, i, i1, i3, r, r1):
            hist(i)
            prod_wait(j, i, r)
            cons_start(j, i, r)
            cons_wait(j - 1, i1, r1)
            _load_wait(tab, j + 1, i1)
            prod_start(j + 1, i1, r1)
            _load_start(tab, j + 3, i3)

        for j in range(3):
            _load_start(tab, j, j)
        _load_wait(tab, 0, 0)
        prod_start(0, 0, 0)
        hist(0)
        prod_wait(0, 0, 0)
        cons_start(0, 0, 0)
        _load_wait(tab, 1, 1)
        prod_start(1, 1, 1)
        _load_start(tab, 3, 3)

        def _quad(jj, carry):
            j0 = 1 + jj * 4
            for k in range(4):
                j = j0 + k
                i, i1, i3 = (1 + k) % NI, (2 + k) % NI, (4 + k) % NI
                r, r1 = (1 + k) % NR, (2 + k) % NR
                _body(j, i, i1, i3, r, r1)
            return carry

        lax.fori_loop(0, 19, _quad, 0)

        hist(1)
        prod_wait(77, 1, 1)
        cons_start(77, 1, 1)
        cons_wait(76, 0, 0)
        _load_wait(tab, 78, 2)
        prod_start(78, 2, 0)
        hist(2)
        prod_wait(78, 2, 0)
        cons_start(78, 2, 0)
        cons_wait(77, 1, 1)
        _load_wait(tab, 79, 3)
        prod_start(79, 3, 1)
        hist(3)
        prod_wait(79, 3, 1)
        cons_start(79, 3, 1)
        cons_wait(78, 2, 0)
        cons_wait(79, 3, 1)

    _phase(src_hbm, _nohist, _ga_start, _ga_wait, _wa_start, _wa_wait)

    plsc.subcore_barrier()

    _zero_slot0()
    base = sid * RPT
    for k in range(4):
        pltpu.sync_copy(rows_v.at[0], big_sh.at[pl.ds(base + k * CH, CH)])
    pltpu.sync_copy(rows_v.at[0].at[pl.ds(0, RPT - 4 * CH)],
                    big_sh.at[pl.ds(base + 4 * CH, RPT - 4 * CH)])

    plsc.subcore_barrier()

    _phase(dst_hbm, _hist_a, _gb_start, _gb_wait, _sb_start, _sb_wait)

    plsc.subcore_barrier()

    pltpu.sync_copy(big_sh.at[pl.ds(base, RPT)],
                    part_hbm.at[cid].at[pl.ds(base, RPT)])
    pltpu.sync_copy(cnt_v, cnt_hbm.at[wid])


def _tc_update(h_ref, p_ref, cnt_ref, w_ref, b_ref, g_ref, be_ref, out_ref):
    h = h_ref[...]
    agg = p_ref[0, 0:N, :] + p_ref[1, 0:N, :]
    cnt = jnp.reshape(jnp.sum(cnt_ref[...], axis=0), (NP, 1))[0:N]
    c = agg / jnp.maximum(cnt, 1.0)
    z = (jnp.dot(h, w_ref[0:D, :], preferred_element_type=jnp.float32)
         + jnp.dot(c, w_ref[D:2 * D, :], preferred_element_type=jnp.float32)
         + b_ref[...])
    nrm = jnp.sqrt(jnp.sum(z * z, axis=1, keepdims=True))
    z = z / jnp.maximum(nrm, 1e-12)
    hout = jnp.maximum(z, 0.0)
    mean = jnp.mean(hout, axis=0, keepdims=True)
    var = jnp.mean(jnp.square(hout - mean), axis=0, keepdims=True)
    out_ref[...] = (h + (hout - mean) * lax.rsqrt(var + 1e-5) * g_ref[...]
                    + be_ref[...])


def kernel(h, edge_index, W, b, gamma, beta):
    pad = EP - E
    src = jnp.concatenate(
        [edge_index[0], jnp.zeros((pad,), jnp.int32)]).reshape(NW, NCH, CH)
    dst = jnp.concatenate(
        [edge_index[1], jnp.full((pad,), N, jnp.int32)]).reshape(NW, NCH, CH)
    part, cnt, _ = _sc_aggregate(h, src, dst)
    out = pl.pallas_call(
        _tc_update,
        out_shape=jax.ShapeDtypeStruct((N, D), jnp.float32),
    )(h, part, cnt, W, b.reshape(1, D), gamma.reshape(1, D), beta.reshape(1, D))
    return out
